# baseline (speedup 1.0000x reference)
import jax
import jax.numpy as jnp
from jax import lax
from jax.experimental import pallas as pl
from jax.experimental.pallas import tpu as pltpu
from jax.experimental.pallas import tpu_sc as plsc

_L = 16
_NC = 2
_NS = 16
_NW = _NC * _NS
_B = 16384
_D = 64
_PER_W = _B // _NW
_C = 32
_NCH = _PER_W // _C


def _sin(x):
    x2 = x * x
    p = jnp.float32(1.0 / 362880.0)
    p = p * x2 + jnp.float32(-1.0 / 5040.0)
    p = p * x2 + jnp.float32(1.0 / 120.0)
    p = p * x2 + jnp.float32(-1.0 / 6.0)
    p = p * x2 + jnp.float32(1.0)
    return x * p


def _body(heads_hbm, rels_hbm, tails_hbm, years_hbm, months_hbm, days_hbm,
          ent_hbm, rel_hbm, *rest):
    tabs = rest[:18]
    out_hbm = rest[18]
    hidx, tidx, ridx = rest[19:22]
    yr_v, mo_v, dy_v = rest[22:25]
    eh_v, et_v, rel_v = rest[25:28]
    gh = rest[28:46]
    gt = rest[46:64]
    p_v, sc_v, sem = rest[64:67]

    wid = lax.axis_index("s") * _NC + lax.axis_index("c")
    base = wid * _PER_W
    lanes = lax.iota(jnp.int32, _L)

    def row(buf, jf, dv):
        return plsc.load_gather(buf, [jf, lanes + dv * _L])

    def chunk_body(c, carry):
        cb = base + c * _C
        pltpu.sync_copy(heads_hbm.at[pl.ds(cb, _C)], hidx)
        pltpu.sync_copy(tails_hbm.at[pl.ds(cb, _C)], tidx)
        pltpu.sync_copy(rels_hbm.at[pl.ds(cb, _C)], ridx)
        pltpu.sync_copy(years_hbm.at[pl.ds(cb, _C)], yr_v.at[0, pl.ds(0, _C)])
        pltpu.sync_copy(months_hbm.at[pl.ds(cb, _C)], mo_v.at[0, pl.ds(0, _C)])
        pltpu.sync_copy(days_hbm.at[pl.ds(cb, _C)], dy_v.at[0, pl.ds(0, _C)])

        handles = [pltpu.async_copy(ent_hbm.at[hidx], eh_v, sem),
                   pltpu.async_copy(ent_hbm.at[tidx], et_v, sem),
                   pltpu.async_copy(rel_hbm.at[ridx], rel_v, sem)]
        for i in range(18):
            handles.append(pltpu.async_copy(tabs[i].at[hidx], gh[i], sem))
            handles.append(pltpu.async_copy(tabs[i].at[tidx], gt[i], sem))
        for h in handles:
            h.wait()

        def elem_body(j, cc):
            jf = jnp.full((_L,), j, jnp.int32)
            z16 = jnp.zeros((_L,), jnp.int32)
            yb = plsc.load_gather(yr_v, [z16, jf])
            mb = plsc.load_gather(mo_v, [z16, jf])
            db = plsc.load_gather(dy_v, [z16, jf])

            def temb(bufs, toff, dv):
                r = None
                for k, tb in enumerate((yb, mb, db)):
                    fq = row(bufs[toff + 3 * k], jf, dv)
                    ph = row(bufs[toff + 3 * k + 1], jf, dv)
                    am = row(bufs[toff + 3 * k + 2], jf, dv)
                    v = am * _sin(fq * tb + ph)
                    r = v if r is None else r + v
                return r

            acc = None
            for dv in range(_D // _L):
                a = row(eh_v, jf, dv)
                b = row(et_v, jf, dv)
                rs = row(rel_v, jf, dv)
                rt = row(rel_v, jf, dv + 4)
                th_h = temb(gh, 0, dv)
                th_t = temb(gt, 0, dv)
                tt_h = temb(gh, 9, dv)
                tt_t = temb(gt, 9, dv)
                v = (a * b * rs
                     + jnp.float32(0.5) * rt * (th_h * tt_t + th_t * tt_h))
                acc = v if acc is None else acc + v
            plsc.store_scatter(p_v, [jf, lanes], acc)
            return cc

        lax.fori_loop(0, _C, elem_body, 0)

        def grp_body(g, cc):
            eidx = lanes + g * _L
            s = jnp.zeros((_L,), jnp.float32)
            for d in range(_L):
                s = s + plsc.load_gather(
                    p_v, [eidx, jnp.full((_L,), d, jnp.int32)])
            sc_v[pl.ds(g * _L, _L)] = s
            return cc

        lax.fori_loop(0, _C // _L, grp_body, 0)
        pltpu.sync_copy(sc_v, out_hbm.at[pl.ds(cb, _C)])
        return carry

    lax.fori_loop(0, _NCH, chunk_body, 0)


@jax.jit
def _run(heads, rels, tails, years, months, days, ent_emb, rel_emb, *tabs):
    mesh = plsc.VectorSubcoreMesh(core_axis_name="c", subcore_axis_name="s")
    fn = pl.kernel(
        _body,
        mesh=mesh,
        out_type=jax.ShapeDtypeStruct((_B,), jnp.float32),
        scratch_types=(
            [pltpu.VMEM((_C,), jnp.int32)] * 3
            + [pltpu.VMEM((1, 128), jnp.float32)] * 3
            + [pltpu.VMEM((_C, _D), jnp.float32)] * 2
            + [pltpu.VMEM((_C, 2 * _D), jnp.float32)]
            + [pltpu.VMEM((_C, _D), jnp.float32)] * 36
            + [pltpu.VMEM((_C, 128), jnp.float32),
               pltpu.VMEM((_C,), jnp.float32),
               pltpu.SemaphoreType.DMA]
        ),
        compiler_params=pltpu.CompilerParams(use_tc_tiling_on_sc=False,
                                             needs_layout_passes=False),
    )
    return fn(heads, rels, tails, years, months, days, ent_emb, rel_emb, *tabs)


def kernel(heads, rels, tails, years, months, days, ent_emb, rel_emb,
           y_freq_h, y_phi_h, y_amps_h, m_freq_h, m_phi_h, m_amps_h,
           d_freq_h, d_phi_h, d_amps_h, y_freq_t, y_phi_t, y_amps_t,
           m_freq_t, m_phi_t, m_amps_t, d_freq_t, d_phi_t, d_amps_t):
    return _run(heads.astype(jnp.int32), rels.astype(jnp.int32),
                tails.astype(jnp.int32), years, months, days, ent_emb, rel_emb,
                y_freq_h, y_phi_h, y_amps_h, m_freq_h, m_phi_h, m_amps_h,
                d_freq_h, d_phi_h, d_amps_h, y_freq_t, y_phi_t, y_amps_t,
                m_freq_t, m_phi_t, m_amps_t, d_freq_t, d_phi_t, d_amps_t)
